# Initial kernel scaffold; baseline (speedup 1.0000x reference)
import jax
import jax.numpy as jnp
from jax.experimental import pallas as pl


def kernel(x, edge_index, W1l, b1l, W1r, W2l, b2l, W2r):
    raise NotImplementedError("write your pallas kernel here")


# speedup vs baseline: 13.7774x; 13.7774x over previous
import functools

import jax
import jax.numpy as jnp
from jax import lax
from jax.experimental import pallas as pl
from jax.experimental.pallas import tpu as pltpu
from jax.experimental.pallas import tpu_sc as plsc

N = 10000
E = 160000
DIM = 256
HID = 512
NCLS = 2

NC = 2
NS = 16
NPAD = 10240
FQ = 64
P16 = 16
NPT = NPAD // NS

EPT_A = E // NS
CH_A = 400
NCH_A = EPT_A // CH_A


def _sc_layer1(xflat, srcq, dst3, ones16, z64, z16, out, cnt_out,
               srcgb, dstb, rowsA, rowsB, onesb, acc, cacc, semA, semB):
    c = lax.axis_index("c")
    s = lax.axis_index("s")

    pltpu.sync_copy(dst3.at[s], dstb)
    pltpu.sync_copy(ones16, onesb)
    pltpu.sync_copy(z64, acc.at[pl.ds(s * NPT, NPT)])
    pltpu.sync_copy(z16, cacc.at[pl.ds(s * NPT, NPT)])
    plsc.subcore_barrier()

    for p in range(2):
        q = 2 * p + c
        pltpu.sync_copy(srcq.at[pl.ds(q * E + s * EPT_A, EPT_A)], srcgb)

        bufs = (rowsA, rowsB)
        sems = (semA, semB)
        hg = {}
        for i in range(2):
            hg[i] = pltpu.async_copy(
                xflat.at[srcgb.at[pl.ds(i * CH_A, CH_A)]], bufs[i], sems[i])
        for i in range(NCH_A):
            buf = bufs[i % 2]
            hg[i].wait()
            pltpu.sync_copy(buf, acc.at[dstb.at[i]], add=True)
            if i + 2 < NCH_A:
                hg[i + 2] = pltpu.async_copy(
                    xflat.at[srcgb.at[pl.ds((i + 2) * CH_A, CH_A)]],
                    buf, sems[i % 2])

        if p == 0:
            for i in range(NCH_A):
                pltpu.sync_copy(onesb, cacc.at[dstb.at[i]], add=True)

        plsc.subcore_barrier()
        pltpu.sync_copy(acc.at[pl.ds(s * NPT, NPT)],
                        out.at[pl.ds(q * NPAD + s * NPT, NPT)])
        if p == 0:
            pltpu.sync_copy(cacc.at[pl.ds(s * NPT, NPT)],
                            cnt_out.at[pl.ds(c * NPAD + s * NPT, NPT)])
            pltpu.sync_copy(z64, acc.at[pl.ds(s * NPT, NPT)])
            plsc.subcore_barrier()


_sc_layer1_call = functools.partial(
    pl.kernel,
    mesh=plsc.VectorSubcoreMesh(core_axis_name="c", subcore_axis_name="s"),
    out_type=[jax.ShapeDtypeStruct((4 * NPAD, FQ), jnp.float32),
              jax.ShapeDtypeStruct((2 * NPAD, P16), jnp.float32)],
    scratch_types=[
        pltpu.VMEM((EPT_A,), jnp.int32),
        pltpu.VMEM((NCH_A, CH_A), jnp.int32),
        pltpu.VMEM((CH_A, FQ), jnp.float32),
        pltpu.VMEM((CH_A, FQ), jnp.float32),
        pltpu.VMEM((CH_A, P16), jnp.float32),
        pltpu.VMEM_SHARED((NPAD, FQ), jnp.float32),
        pltpu.VMEM_SHARED((NPAD, P16), jnp.float32),
        pltpu.SemaphoreType.DMA,
        pltpu.SemaphoreType.DMA,
    ],
    compiler_params=pltpu.CompilerParams(use_tc_tiling_on_sc=False),
)(_sc_layer1)

EPT_B = E // (NC * NS)
CH_B = 1000
NCH_B = EPT_B // CH_B


def _sc_layer2(p16, src3, dst3, z16, out, srcb, dstb, rowsA, rowsB, acc,
               semA, semB):
    c = lax.axis_index("c")
    s = lax.axis_index("s")
    w = c * NS + s

    pltpu.sync_copy(src3.at[w], srcb)
    pltpu.sync_copy(dst3.at[w], dstb)
    pltpu.sync_copy(z16, acc.at[pl.ds(s * NPT, NPT)])
    plsc.subcore_barrier()

    bufs = (rowsA, rowsB)
    sems = (semA, semB)
    hg = {}
    for i in range(2):
        hg[i] = pltpu.async_copy(p16.at[srcb.at[i]], bufs[i], sems[i])
    for i in range(NCH_B):
        buf = bufs[i % 2]
        hg[i].wait()
        pltpu.sync_copy(buf, acc.at[dstb.at[i]], add=True)
        if i + 2 < NCH_B:
            hg[i + 2] = pltpu.async_copy(p16.at[srcb.at[i + 2]], buf,
                                         sems[i % 2])

    plsc.subcore_barrier()
    pltpu.sync_copy(acc.at[pl.ds(s * NPT, NPT)],
                    out.at[pl.ds(c * NPAD + s * NPT, NPT)])


_sc_layer2_call = functools.partial(
    pl.kernel,
    mesh=plsc.VectorSubcoreMesh(core_axis_name="c", subcore_axis_name="s"),
    out_type=jax.ShapeDtypeStruct((2 * NPAD, P16), jnp.float32),
    scratch_types=[
        pltpu.VMEM((NCH_B, CH_B), jnp.int32),
        pltpu.VMEM((NCH_B, CH_B), jnp.int32),
        pltpu.VMEM((CH_B, P16), jnp.float32),
        pltpu.VMEM((CH_B, P16), jnp.float32),
        pltpu.VMEM_SHARED((NPAD, P16), jnp.float32),
        pltpu.SemaphoreType.DMA,
        pltpu.SemaphoreType.DMA,
    ],
    compiler_params=pltpu.CompilerParams(use_tc_tiling_on_sc=False),
)(_sc_layer2)

BM = 1024


def _tc_hidden(s0, s1, s2, s3, cnt, x, a, b, b1, w2, p16_out):
    inv = 1.0 / jnp.maximum(cnt[:, 0:1], 1.0)
    z = (jnp.dot(s0[0] * inv, a[0 * FQ:1 * FQ, :],
                 preferred_element_type=jnp.float32)
         + jnp.dot(s1[0] * inv, a[1 * FQ:2 * FQ, :],
                   preferred_element_type=jnp.float32)
         + jnp.dot(s2[0] * inv, a[2 * FQ:3 * FQ, :],
                   preferred_element_type=jnp.float32)
         + jnp.dot(s3[0] * inv, a[3 * FQ:4 * FQ, :],
                   preferred_element_type=jnp.float32)
         + jnp.dot(x[...], b[...], preferred_element_type=jnp.float32)
         + b1[...])
    h = jnp.maximum(z, 0.0)
    p16_out[...] = jnp.dot(h, w2[...], preferred_element_type=jnp.float32)


def _quarter_spec(q):
    return pl.BlockSpec((1, BM, FQ), lambda i, _q=q: (_q, i, 0))


def _tc_hidden_call(summed4, cnt, x, a, b, b1, w2):
    grid = (NPAD // BM,)
    return pl.pallas_call(
        _tc_hidden,
        grid=grid,
        in_specs=[
            _quarter_spec(0), _quarter_spec(1), _quarter_spec(2),
            _quarter_spec(3),
            pl.BlockSpec((BM, P16), lambda i: (i, 0)),
            pl.BlockSpec((BM, DIM), lambda i: (i, 0)),
            pl.BlockSpec((DIM, HID), lambda i: (0, 0)),
            pl.BlockSpec((DIM, HID), lambda i: (0, 0)),
            pl.BlockSpec((1, HID), lambda i: (0, 0)),
            pl.BlockSpec((HID, P16), lambda i: (0, 0)),
        ],
        out_specs=pl.BlockSpec((BM, P16), lambda i: (i, 0)),
        out_shape=jax.ShapeDtypeStruct((N, P16), jnp.float32),
    )(summed4, summed4, summed4, summed4, cnt, x, a, b, b1, w2)


def _tc_out(agga, aggb, cnt, p16, b2, out):
    inv = 1.0 / jnp.maximum(cnt[:, 0:1], 1.0)
    mean2 = (agga[0][:, 0:NCLS] + aggb[0][:, 0:NCLS]) * inv
    out[...] = mean2 + b2[...] + p16[:, NCLS:2 * NCLS]


def _tc_out_call(agg2, cnt, p16, b2):
    grid = (NPAD // BM,)
    return pl.pallas_call(
        _tc_out,
        grid=grid,
        in_specs=[
            pl.BlockSpec((1, BM, P16), lambda i: (0, i, 0)),
            pl.BlockSpec((1, BM, P16), lambda i: (1, i, 0)),
            pl.BlockSpec((BM, P16), lambda i: (i, 0)),
            pl.BlockSpec((BM, P16), lambda i: (i, 0)),
            pl.BlockSpec((1, NCLS), lambda i: (0, 0)),
        ],
        out_specs=pl.BlockSpec((BM, NCLS), lambda i: (i, 0)),
        out_shape=jax.ShapeDtypeStruct((N, NCLS), jnp.float32),
    )(agg2, agg2, cnt, p16, b2)


def kernel(x, edge_index, W1l, b1l, W1r, W2l, b2l, W2r):
    src = edge_index[0].astype(jnp.int32)
    dst = edge_index[1].astype(jnp.int32)

    xflat = x.reshape(4 * N, FQ)
    srcq = (src[None, :] * 4 + jnp.arange(4, dtype=jnp.int32)[:, None]).ravel()
    dst3a = dst.reshape(NS, NCH_A, CH_A)
    ones16 = jnp.ones((CH_A, P16), jnp.float32)
    z64 = jnp.zeros((NPT, FQ), jnp.float32)
    z16 = jnp.zeros((NPT, P16), jnp.float32)

    summed, cnt = _sc_layer1_call(xflat, srcq, dst3a, ones16, z64, z16)
    summed4 = summed.reshape(4, NPAD, FQ)

    a = W1l.T
    b = W1r.T
    w2 = jnp.concatenate(
        [W2l.T, W2r.T, jnp.zeros((HID, P16 - 2 * NCLS), jnp.float32)], axis=1)
    p16 = _tc_hidden_call(summed4, cnt, x, a, b, b1l.reshape(1, HID), w2)

    src3b = src.reshape(NC * NS, NCH_B, CH_B)
    dst3b = dst.reshape(NC * NS, NCH_B, CH_B)
    agg2 = _sc_layer2_call(p16, src3b, dst3b, z16)

    return _tc_out_call(agg2.reshape(2, NPAD, P16), cnt, p16,
                        b2l.reshape(1, NCLS))
